# baseline (speedup 1.0000x reference)
import functools

import jax
import jax.numpy as jnp
from jax import lax
from jax.experimental import pallas as pl
from jax.experimental.pallas import tpu as pltpu
from jax.experimental.pallas import tpu_sc as plsc

_BATCH = 16384
_D = 64
_NC = 2
_NS = 16
_NW = _NC * _NS
_BPW = _BATCH // _NW
_CH = 128
_NCH = _BPW // _CH


def _make_gather():
    mesh = plsc.VectorSubcoreMesh(core_axis_name="c", subcore_axis_name="s")

    @functools.partial(
        pl.kernel,
        mesh=mesh,
        out_type=(
            jax.ShapeDtypeStruct((_BATCH, _D), jnp.float32),
            jax.ShapeDtypeStruct((_BATCH, _D), jnp.float32),
        ),
        scratch_types=[
            pltpu.VMEM((_NCH, _CH), jnp.int32),
            pltpu.VMEM((_NCH, _CH), jnp.int32),
            pltpu.VMEM((_BPW, _D), jnp.float32),
            pltpu.VMEM((_BPW, _D), jnp.float32),
            pltpu.SemaphoreType.DMA,
        ],
        compiler_params=pltpu.CompilerParams(use_tc_tiling_on_sc=False),
    )
    def gather2(uid_hbm, iid_hbm, ut_hbm, it_hbm, uout_hbm, iout_hbm,
                uidx, iidx, urows, irows, sem):
        wid = lax.axis_index("s") * _NC + lax.axis_index("c")
        base = wid * _BPW
        for j in range(_NCH):
            pltpu.sync_copy(uid_hbm.at[pl.ds(base + j * _CH, _CH)], uidx.at[j])
            pltpu.sync_copy(iid_hbm.at[pl.ds(base + j * _CH, _CH)], iidx.at[j])
        copies = []
        for j in range(_NCH):
            copies.append(pltpu.async_copy(
                ut_hbm.at[uidx.at[j]], urows.at[pl.ds(j * _CH, _CH)], sem))
            copies.append(pltpu.async_copy(
                it_hbm.at[iidx.at[j]], irows.at[pl.ds(j * _CH, _CH)], sem))
        for c in copies:
            c.wait()
        pltpu.sync_copy(urows, uout_hbm.at[pl.ds(base, _BPW)])
        pltpu.sync_copy(irows, iout_hbm.at[pl.ds(base, _BPW)])

    return gather2


_gather2 = _make_gather()

_BLK = 2048


def _mlp_body(u_ref, v_ref, w1u_ref, w1v_ref, b1_ref, w2_ref, b2_ref,
              w3_ref, b3_ref, o_ref):
    h = jnp.dot(u_ref[...], w1u_ref[...], preferred_element_type=jnp.float32)
    h = h + jnp.dot(v_ref[...], w1v_ref[...], preferred_element_type=jnp.float32)
    h = jnp.maximum(h + b1_ref[...], 0.0)
    h = jnp.maximum(
        jnp.dot(h, w2_ref[...], preferred_element_type=jnp.float32) + b2_ref[...],
        0.0)
    o_ref[...] = jnp.dot(h, w3_ref[...], preferred_element_type=jnp.float32) + b3_ref[...]


def _mlp(u, v, w1u, w1v, b1, w2, b2, w3, b3):
    full = lambda i: (0, 0)
    return pl.pallas_call(
        _mlp_body,
        grid=(_BATCH // _BLK,),
        in_specs=[
            pl.BlockSpec((_BLK, _D), lambda i: (i, 0)),
            pl.BlockSpec((_BLK, _D), lambda i: (i, 0)),
            pl.BlockSpec((_D, 64), full),
            pl.BlockSpec((_D, 64), full),
            pl.BlockSpec((1, 64), full),
            pl.BlockSpec((64, 32), full),
            pl.BlockSpec((1, 32), full),
            pl.BlockSpec((32, 1), full),
            pl.BlockSpec((1, 1), full),
        ],
        out_specs=pl.BlockSpec((_BLK, 1), lambda i: (i, 0)),
        out_shape=jax.ShapeDtypeStruct((_BATCH, 1), jnp.float32),
    )(u, v, w1u, w1v, b1, w2, b2, w3, b3)


def kernel(user_id, item_id, user_table, item_table, W1, b1, W2, b2, W3, b3):
    u, v = _gather2(user_id.astype(jnp.int32), item_id.astype(jnp.int32),
                    user_table, item_table)
    return _mlp(u, v,
                W1[:_D], W1[_D:], b1.reshape(1, 64),
                W2, b2.reshape(1, 32),
                W3, b3.reshape(1, 1))


# speedup vs baseline: 1.5771x; 1.5771x over previous
import functools

import jax
import jax.numpy as jnp
from jax import lax
from jax.experimental import pallas as pl
from jax.experimental.pallas import tpu as pltpu
from jax.experimental.pallas import tpu_sc as plsc

_BATCH = 16384
_D = 64
_NC = 2
_NS = 16
_NW = _NC * _NS
_BPW = _BATCH // _NW
_PASS = 256


def _make_gather():
    mesh = plsc.VectorSubcoreMesh(core_axis_name="c", subcore_axis_name="s")

    @functools.partial(
        pl.kernel,
        mesh=mesh,
        out_type=(
            jax.ShapeDtypeStruct((_BATCH, _D), jnp.float32),
            jax.ShapeDtypeStruct((_BATCH, _D), jnp.float32),
        ),
        scratch_types=[
            pltpu.VMEM((_BPW,), jnp.int32),
            pltpu.VMEM((_BPW,), jnp.int32),
            pltpu.VMEM((_PASS, _D), jnp.float32),
            pltpu.VMEM((_PASS, _D), jnp.float32),
            pltpu.SemaphoreType.DMA,
        ],
    )
    def gather2(uid_hbm, iid_hbm, ut_hbm, it_hbm, u_hbm, v_hbm,
                uidx, iidx, urows, irows, sem):
        wid = lax.axis_index("s") * _NC + lax.axis_index("c")
        base = wid * _BPW
        pltpu.sync_copy(uid_hbm.at[pl.ds(base, _BPW)], uidx)
        pltpu.sync_copy(iid_hbm.at[pl.ds(base, _BPW)], iidx)

        for p in range(_BPW // _PASS):
            def body(j, carry):
                b = p * _PASS + j * 16
                uvec = uidx[pl.ds(b, 16)]
                ivec = iidx[pl.ds(b, 16)]
                r = j * 16
                for k in range(16):
                    pltpu.async_copy(ut_hbm.at[pl.ds(uvec[k], 1)],
                                     urows.at[pl.ds(r + k, 1)], sem)
                    pltpu.async_copy(it_hbm.at[pl.ds(ivec[k], 1)],
                                     irows.at[pl.ds(r + k, 1)], sem)
                return carry

            lax.fori_loop(0, _PASS // 16, body, 0)
            pltpu.make_async_copy(
                u_hbm.at[pl.ds(base + p * _PASS, _PASS)], urows, sem).wait()
            pltpu.make_async_copy(
                v_hbm.at[pl.ds(base + p * _PASS, _PASS)], irows, sem).wait()
            pltpu.sync_copy(urows, u_hbm.at[pl.ds(base + p * _PASS, _PASS)])
            pltpu.sync_copy(irows, v_hbm.at[pl.ds(base + p * _PASS, _PASS)])

    return gather2


_gather2 = _make_gather()

_BLK = 2048


def _mlp_body(u_ref, v_ref, w1u_ref, w1v_ref, b1_ref, w2_ref, b2_ref,
              w3_ref, b3_ref, o_ref):
    h = jnp.dot(u_ref[...], w1u_ref[...], preferred_element_type=jnp.float32)
    h = h + jnp.dot(v_ref[...], w1v_ref[...], preferred_element_type=jnp.float32)
    h = jnp.maximum(h + b1_ref[...], 0.0)
    h = jnp.maximum(
        jnp.dot(h, w2_ref[...], preferred_element_type=jnp.float32) + b2_ref[...],
        0.0)
    o_ref[...] = jnp.dot(h, w3_ref[...], preferred_element_type=jnp.float32) + b3_ref[...]


def _mlp(u, v, w1u, w1v, b1, w2, b2, w3, b3):
    full = lambda i: (0, 0)
    return pl.pallas_call(
        _mlp_body,
        grid=(_BATCH // _BLK,),
        in_specs=[
            pl.BlockSpec((_BLK, _D), lambda i: (i, 0)),
            pl.BlockSpec((_BLK, _D), lambda i: (i, 0)),
            pl.BlockSpec((_D, 64), full),
            pl.BlockSpec((_D, 64), full),
            pl.BlockSpec((1, 64), full),
            pl.BlockSpec((64, 32), full),
            pl.BlockSpec((1, 32), full),
            pl.BlockSpec((32, 1), full),
            pl.BlockSpec((1, 1), full),
        ],
        out_specs=pl.BlockSpec((_BLK, 1), lambda i: (i, 0)),
        out_shape=jax.ShapeDtypeStruct((_BATCH, 1), jnp.float32),
    )(u, v, w1u, w1v, b1, w2, b2, w3, b3)


def kernel(user_id, item_id, user_table, item_table, W1, b1, W2, b2, W3, b3):
    u, v = _gather2(user_id.astype(jnp.int32), item_id.astype(jnp.int32),
                    user_table, item_table)
    return _mlp(u, v,
                W1[:_D], W1[_D:], b1.reshape(1, 64),
                W2, b2.reshape(1, 32),
                W3, b3.reshape(1, 1))
